# Initial kernel scaffold; baseline (speedup 1.0000x reference)
import jax
import jax.numpy as jnp
from jax.experimental import pallas as pl


def kernel(img):
    raise NotImplementedError("write your pallas kernel here")


# speedup vs baseline: 2.8369x; 2.8369x over previous
import functools

import numpy as np
import jax
import jax.numpy as jnp
from jax.experimental import pallas as pl
from jax.experimental.pallas import tpu as pltpu

_MIN_DISABLED = 4
_MAX_DISABLED = 16


def _disabled_tofs(tof_count: int) -> np.ndarray:
    rng = np.random.RandomState(0)
    disabled_count = int(rng.randint(_MIN_DISABLED, _MAX_DISABLED + 1))
    initial = int(rng.randint(0, tof_count))
    disabled = [initial]
    tof_list = rng.permutation(tof_count)
    tof_list = tof_list[tof_list != initial]
    for _ in range(disabled_count - 1):
        perm = rng.permutation(len(disabled))
        permuted = [disabled[i] for i in perm]
        opposite_found = False
        for cur in permuted:
            new_opp = (cur + tof_count // 2) % tof_count
            if new_opp not in disabled:
                disabled.append(int(new_opp))
                tof_list = tof_list[tof_list != new_opp]
                opposite_found = True
                break
        if not opposite_found:
            new_el = int(tof_list[0])
            tof_list = tof_list[tof_list != new_el]
            disabled.append(new_el)
    return np.asarray(disabled, dtype=np.int64)


def _masked_copy_body(x_ref, m_ref, o_ref):
    o_ref[...] = x_ref[...] * m_ref[0:1, :]


@functools.partial(jax.jit, static_argnames=("block_rows",))
def _masked_copy(img, mask, block_rows=2048):
    rows, cols = img.shape
    grid = rows // block_rows
    return pl.pallas_call(
        _masked_copy_body,
        grid=(grid,),
        in_specs=[
            pl.BlockSpec((block_rows, cols), lambda i: (i, 0)),
            pl.BlockSpec((8, cols), lambda i: (0, 0)),
        ],
        out_specs=pl.BlockSpec((block_rows, cols), lambda i: (i, 0)),
        out_shape=jax.ShapeDtypeStruct((rows, cols), img.dtype),
        compiler_params=pltpu.CompilerParams(
            dimension_semantics=("arbitrary",),
        ),
    )(img, mask)


def kernel(img) -> jnp.ndarray:
    tof_count = img.shape[-1]
    disabled = _disabled_tofs(tof_count)
    mask = np.ones((8, tof_count), dtype=np.float32)
    mask[:, disabled] = 0.0
    return _masked_copy(img, jnp.asarray(mask))
